# Initial kernel scaffold; baseline (speedup 1.0000x reference)
import jax
import jax.numpy as jnp
from jax.experimental import pallas as pl


def kernel(x, edge_index, W1, b1, W2, b2):
    raise NotImplementedError("write your pallas kernel here")


# speedup vs baseline: 17.6754x; 17.6754x over previous
import functools

import jax
import jax.numpy as jnp
from jax import lax
from jax.experimental import pallas as pl
from jax.experimental.pallas import tpu as pltpu
from jax.experimental.pallas import tpu_sc as plsc

N = 10000
NP = 10240
E = 320000
NW = 32
CHUNK = 128
CHUNKS = 80
EPW = CHUNK * CHUNKS
EP = EPW * NW
RPT = NP // 16

_mesh = plsc.VectorSubcoreMesh(
    core_axis_name="c", subcore_axis_name="s", num_cores=2, num_subcores=16)


@functools.partial(
    pl.kernel,
    out_type=jax.ShapeDtypeStruct((2 * NP,), jnp.float32),
    mesh=_mesh,
    scratch_types=[
        pltpu.VMEM((CHUNKS, CHUNK), jnp.int32),
        pltpu.VMEM((CHUNK,), jnp.float32),
        pltpu.VMEM((RPT,), jnp.float32),
        pltpu.VMEM_SHARED((NP,), jnp.float32),
        pltpu.SemaphoreType.DMA,
    ],
)
def _deg_kernel(dst_hbm, zeros_hbm, out_hbm, idx_v, ones_v, stage_v, acc_sh, sem):
    c = lax.axis_index("c")
    s = lax.axis_index("s")
    wid = s * 2 + c
    pltpu.sync_copy(zeros_hbm.at[pl.ds(s * RPT, RPT)], stage_v)
    pltpu.sync_copy(stage_v, acc_sh.at[pl.ds(s * RPT, RPT)])
    for i in range(CHUNK // 16):
        ones_v[pl.ds(i * 16, 16)] = jnp.ones((16,), jnp.float32)
    pltpu.sync_copy(dst_hbm.at[pl.ds(wid * CHUNKS, CHUNKS)], idx_v)
    plsc.subcore_barrier()

    def body(j, carry):
        pltpu.sync_copy(ones_v, acc_sh.at[idx_v.at[j]], add=True)
        return carry

    lax.fori_loop(0, CHUNKS, body, 0)
    plsc.subcore_barrier()
    pltpu.sync_copy(acc_sh.at[pl.ds(s * RPT, RPT)], stage_v)
    pltpu.sync_copy(stage_v, out_hbm.at[pl.ds(c * NP + s * RPT, RPT)])


def _make_agg(D):
    @functools.partial(
        pl.kernel,
        out_type=jax.ShapeDtypeStruct((2 * NP, D), jnp.float32),
        mesh=_mesh,
        compiler_params=pltpu.CompilerParams(use_tc_tiling_on_sc=False),
        scratch_types=[
            pltpu.VMEM((CHUNKS, CHUNK), jnp.int32),
            pltpu.VMEM((CHUNKS, CHUNK), jnp.int32),
            pltpu.VMEM((CHUNK, D), jnp.float32),
            pltpu.VMEM((RPT, D), jnp.float32),
            pltpu.VMEM_SHARED((NP, D), jnp.float32),
            pltpu.SemaphoreType.DMA,
        ],
    )
    def _agg(g_hbm, src_hbm, dst_hbm, zeros_hbm, out_hbm,
             idxs_v, idxd_v, rows_v, stage_v, acc_sh, sem):
        c = lax.axis_index("c")
        s = lax.axis_index("s")
        wid = s * 2 + c
        pltpu.sync_copy(zeros_hbm.at[pl.ds(s * RPT, RPT)], stage_v)
        pltpu.sync_copy(stage_v, acc_sh.at[pl.ds(s * RPT, RPT)])
        pltpu.sync_copy(src_hbm.at[pl.ds(wid * CHUNKS, CHUNKS)], idxs_v)
        pltpu.sync_copy(dst_hbm.at[pl.ds(wid * CHUNKS, CHUNKS)], idxd_v)
        plsc.subcore_barrier()

        def body(j, carry):
            pltpu.async_copy(g_hbm.at[idxs_v.at[j]], rows_v, sem).wait()
            pltpu.sync_copy(rows_v, acc_sh.at[idxd_v.at[j]], add=True)
            return carry

        lax.fori_loop(0, CHUNKS, body, 0)
        plsc.subcore_barrier()
        pltpu.sync_copy(acc_sh.at[pl.ds(s * RPT, RPT)], stage_v)
        pltpu.sync_copy(stage_v, out_hbm.at[pl.ds(c * NP + s * RPT, RPT)])

    return _agg


_agg64 = _make_agg(64)
_agg16 = _make_agg(16)


_B = 2048


def _tc1_body(x_ref, w1_ref, degp_ref, g1_ref, dinv_ref):
    deg = degp_ref[0] + degp_ref[1] + 1.0
    dinv = lax.rsqrt(deg)
    h = jnp.dot(x_ref[...], w1_ref[...], preferred_element_type=jnp.float32)
    g1_ref[...] = h * dinv
    dinv_ref[...] = dinv


def _tc1(x_p, W1, degp3):
    return pl.pallas_call(
        _tc1_body,
        grid=(NP // _B,),
        in_specs=[
            pl.BlockSpec((_B, 128), lambda i: (i, 0)),
            pl.BlockSpec((128, 64), lambda i: (0, 0)),
            pl.BlockSpec((2, _B, 1), lambda i: (0, i, 0)),
        ],
        out_specs=[
            pl.BlockSpec((_B, 64), lambda i: (i, 0)),
            pl.BlockSpec((_B, 1), lambda i: (i, 0)),
        ],
        out_shape=[
            jax.ShapeDtypeStruct((NP, 64), jnp.float32),
            jax.ShapeDtypeStruct((NP, 1), jnp.float32),
        ],
    )(x_p, W1, degp3)


def _tc2_body(p1_ref, g1_ref, dinv_ref, b1_ref, w2_ref, g2_ref):
    ssum = p1_ref[0] + p1_ref[1] + g1_ref[...]
    out1 = ssum * dinv_ref[...] + b1_ref[...]
    r = jnp.maximum(out1, 0.0)
    h2 = jnp.dot(r, w2_ref[...], preferred_element_type=jnp.float32)
    g2_ref[...] = h2 * dinv_ref[...]


def _tc2(p1, g1, dinv, b1r, W2):
    return pl.pallas_call(
        _tc2_body,
        grid=(NP // _B,),
        in_specs=[
            pl.BlockSpec((2, _B, 64), lambda i: (0, i, 0)),
            pl.BlockSpec((_B, 64), lambda i: (i, 0)),
            pl.BlockSpec((_B, 1), lambda i: (i, 0)),
            pl.BlockSpec((1, 64), lambda i: (0, 0)),
            pl.BlockSpec((64, 16), lambda i: (0, 0)),
        ],
        out_specs=pl.BlockSpec((_B, 16), lambda i: (i, 0)),
        out_shape=jax.ShapeDtypeStruct((NP, 16), jnp.float32),
    )(p1, g1, dinv, b1r, W2)


def _tc3_body(p2_ref, g2_ref, dinv_ref, b2_ref, out_ref):
    ssum = p2_ref[0] + p2_ref[1] + g2_ref[...]
    out_ref[...] = ssum[:, :2] * dinv_ref[...] + b2_ref[...]


def _tc3(p2, g2, dinv, b2r):
    return pl.pallas_call(
        _tc3_body,
        grid=(NP // _B,),
        in_specs=[
            pl.BlockSpec((2, _B, 16), lambda i: (0, i, 0)),
            pl.BlockSpec((_B, 16), lambda i: (i, 0)),
            pl.BlockSpec((_B, 1), lambda i: (i, 0)),
            pl.BlockSpec((1, 2), lambda i: (0, 0)),
        ],
        out_specs=pl.BlockSpec((_B, 2), lambda i: (i, 0)),
        out_shape=jax.ShapeDtypeStruct((NP, 2), jnp.float32),
    )(p2, g2, dinv, b2r)


def kernel(x, edge_index, W1, b1, W2, b2):
    src = edge_index[0].astype(jnp.int32)
    dst = edge_index[1].astype(jnp.int32)
    pad = jnp.full((EP - E,), N, jnp.int32)
    src_m = jnp.concatenate([src, pad]).reshape(EP // CHUNK, CHUNK)
    dst_m = jnp.concatenate([dst, pad]).reshape(EP // CHUNK, CHUNK)
    x_p = jnp.pad(x, ((0, NP - N), (0, 0)))

    zeros1 = jnp.zeros((NP,), jnp.float32)
    zeros64 = jnp.zeros((NP, 64), jnp.float32)
    zeros16 = jnp.zeros((NP, 16), jnp.float32)
    W2p = jnp.pad(W2, ((0, 0), (0, 16 - 2)))

    degp = _deg_kernel(dst_m, zeros1)
    degp3 = degp.reshape(2, NP, 1)
    g1, dinv = _tc1(x_p, W1, degp3)
    p1 = _agg64(g1, src_m, dst_m, zeros64).reshape(2, NP, 64)
    g2 = _tc2(p1, g1, dinv, b1.reshape(1, 64), W2p)
    p2 = _agg16(g2, src_m, dst_m, zeros16).reshape(2, NP, 16)
    out = _tc3(p2, g2, dinv, b2.reshape(1, 2))
    return out[:N]


# speedup vs baseline: 20.6315x; 1.1672x over previous
import functools

import jax
import jax.numpy as jnp
from jax import lax
from jax.experimental import pallas as pl
from jax.experimental.pallas import tpu as pltpu
from jax.experimental.pallas import tpu_sc as plsc

N = 10000
NP = 10240
E = 320000
NW = 32
CHUNK = 128
CHUNKS = 80
EPW = CHUNK * CHUNKS
EP = EPW * NW
RPT = NP // 16

_mesh = plsc.VectorSubcoreMesh(
    core_axis_name="c", subcore_axis_name="s", num_cores=2, num_subcores=16)


@functools.partial(
    pl.kernel,
    out_type=jax.ShapeDtypeStruct((2 * NP,), jnp.float32),
    mesh=_mesh,
    scratch_types=[
        pltpu.VMEM((CHUNKS, CHUNK), jnp.int32),
        pltpu.VMEM((CHUNK,), jnp.float32),
        pltpu.VMEM((RPT,), jnp.float32),
        pltpu.VMEM_SHARED((NP,), jnp.float32),
        pltpu.SemaphoreType.DMA,
    ],
)
def _deg_kernel(dst_hbm, zeros_hbm, out_hbm, idx_v, ones_v, stage_v, acc_sh, sem):
    c = lax.axis_index("c")
    s = lax.axis_index("s")
    wid = s * 2 + c
    pltpu.sync_copy(zeros_hbm.at[pl.ds(s * RPT, RPT)], stage_v)
    pltpu.sync_copy(stage_v, acc_sh.at[pl.ds(s * RPT, RPT)])
    for i in range(CHUNK // 16):
        ones_v[pl.ds(i * 16, 16)] = jnp.ones((16,), jnp.float32)
    pltpu.sync_copy(dst_hbm.at[pl.ds(wid * CHUNKS, CHUNKS)], idx_v)
    plsc.subcore_barrier()

    NB = 8
    for b in range(NB):
        pltpu.async_copy(ones_v, acc_sh.at[idx_v.at[b]], sem, add=True)

    def body(j, carry):
        pltpu.async_copy(ones_v, acc_sh.at[idx_v.at[j + NB]], sem, add=True)
        pltpu.make_async_copy(ones_v, acc_sh.at[idx_v.at[j]], sem).wait()
        return carry

    lax.fori_loop(0, CHUNKS - NB, body, 0)
    for b in range(NB):
        pltpu.make_async_copy(ones_v, acc_sh.at[idx_v.at[b]], sem).wait()
    plsc.subcore_barrier()
    pltpu.sync_copy(acc_sh.at[pl.ds(s * RPT, RPT)], stage_v)
    pltpu.sync_copy(stage_v, out_hbm.at[pl.ds(c * NP + s * RPT, RPT)])


def _make_agg(D, NB):
    @functools.partial(
        pl.kernel,
        out_type=jax.ShapeDtypeStruct((2 * NP, D), jnp.float32),
        mesh=_mesh,
        compiler_params=pltpu.CompilerParams(use_tc_tiling_on_sc=False),
        scratch_types=[
            pltpu.VMEM((CHUNKS, CHUNK), jnp.int32),
            pltpu.VMEM((CHUNKS, CHUNK), jnp.int32),
            pltpu.VMEM((NB, CHUNK, D), jnp.float32),
            pltpu.VMEM((NB, CHUNK, D), jnp.float32),
            pltpu.VMEM_SHARED((NP, D), jnp.float32),
            pltpu.SemaphoreType.DMA,
            pltpu.SemaphoreType.DMA,
        ],
    )
    def _agg(g_hbm, src_hbm, dst_hbm, zeros_hbm, out_hbm,
             idxs_v, idxd_v, rows_a, rows_b, acc_sh, gsem, ssem):
        c = lax.axis_index("c")
        s = lax.axis_index("s")
        wid = s * 2 + c
        for i in range(RPT // CHUNK):
            pltpu.sync_copy(zeros_hbm.at[pl.ds(s * RPT + i * CHUNK, CHUNK)],
                            rows_a.at[0])
            pltpu.sync_copy(rows_a.at[0],
                            acc_sh.at[pl.ds(s * RPT + i * CHUNK, CHUNK)])
        pltpu.sync_copy(src_hbm.at[pl.ds(wid * CHUNKS, CHUNKS)], idxs_v)
        pltpu.sync_copy(dst_hbm.at[pl.ds(wid * CHUNKS, CHUNKS)], idxd_v)
        plsc.subcore_barrier()

        def gstart(j, buf, b):
            pltpu.async_copy(g_hbm.at[idxs_v.at[j + b]], buf.at[b], gsem)

        def gwait(j, buf, b):
            pltpu.make_async_copy(g_hbm.at[idxs_v.at[j + b]], buf.at[b],
                                  gsem).wait()

        def sstart(j, buf, b):
            pltpu.async_copy(buf.at[b], acc_sh.at[idxd_v.at[j + b]], ssem,
                             add=True)

        def swait(j, buf, b):
            pltpu.make_async_copy(buf.at[b], acc_sh.at[idxd_v.at[j + b]],
                                  ssem).wait()

        nit = CHUNKS // (2 * NB)
        for b in range(NB):
            gstart(0, rows_a, b)

        def body(k, carry):
            ja = 2 * NB * k
            jb = ja + NB
            for b in range(NB):
                gwait(ja, rows_a, b)
            for b in range(NB):
                sstart(ja, rows_a, b)
            for b in range(NB):
                gstart(jb, rows_b, b)
            for b in range(NB):
                swait(ja, rows_a, b)
            for b in range(NB):
                gwait(jb, rows_b, b)
            for b in range(NB):
                sstart(jb, rows_b, b)

            @pl.when(k < nit - 1)
            def _():
                for b in range(NB):
                    gstart(ja + 2 * NB, rows_a, b)

            for b in range(NB):
                swait(jb, rows_b, b)
            return carry

        lax.fori_loop(0, nit, body, 0)
        plsc.subcore_barrier()
        for i in range(RPT // CHUNK):
            pltpu.sync_copy(acc_sh.at[pl.ds(s * RPT + i * CHUNK, CHUNK)],
                            rows_a.at[0])
            pltpu.sync_copy(rows_a.at[0],
                            out_hbm.at[pl.ds(c * NP + s * RPT + i * CHUNK,
                                             CHUNK)])

    return _agg


_agg64 = _make_agg(64, 4)
_agg16 = _make_agg(16, 8)


_B = 2048


def _tc1_body(x_ref, w1_ref, degp_ref, g1_ref, dinv_ref):
    deg = degp_ref[0] + degp_ref[1] + 1.0
    dinv = lax.rsqrt(deg)
    h = jnp.dot(x_ref[...], w1_ref[...], preferred_element_type=jnp.float32)
    g1_ref[...] = h * dinv
    dinv_ref[...] = dinv


def _tc1(x_p, W1, degp3):
    return pl.pallas_call(
        _tc1_body,
        grid=(NP // _B,),
        in_specs=[
            pl.BlockSpec((_B, 128), lambda i: (i, 0)),
            pl.BlockSpec((128, 64), lambda i: (0, 0)),
            pl.BlockSpec((2, _B, 1), lambda i: (0, i, 0)),
        ],
        out_specs=[
            pl.BlockSpec((_B, 64), lambda i: (i, 0)),
            pl.BlockSpec((_B, 1), lambda i: (i, 0)),
        ],
        out_shape=[
            jax.ShapeDtypeStruct((NP, 64), jnp.float32),
            jax.ShapeDtypeStruct((NP, 1), jnp.float32),
        ],
    )(x_p, W1, degp3)


def _tc2_body(p1_ref, g1_ref, dinv_ref, b1_ref, w2_ref, g2_ref):
    ssum = p1_ref[0] + p1_ref[1] + g1_ref[...]
    out1 = ssum * dinv_ref[...] + b1_ref[...]
    r = jnp.maximum(out1, 0.0)
    h2 = jnp.dot(r, w2_ref[...], preferred_element_type=jnp.float32)
    g2_ref[...] = h2 * dinv_ref[...]


def _tc2(p1, g1, dinv, b1r, W2):
    return pl.pallas_call(
        _tc2_body,
        grid=(NP // _B,),
        in_specs=[
            pl.BlockSpec((2, _B, 64), lambda i: (0, i, 0)),
            pl.BlockSpec((_B, 64), lambda i: (i, 0)),
            pl.BlockSpec((_B, 1), lambda i: (i, 0)),
            pl.BlockSpec((1, 64), lambda i: (0, 0)),
            pl.BlockSpec((64, 16), lambda i: (0, 0)),
        ],
        out_specs=pl.BlockSpec((_B, 16), lambda i: (i, 0)),
        out_shape=jax.ShapeDtypeStruct((NP, 16), jnp.float32),
    )(p1, g1, dinv, b1r, W2)


def _tc3_body(p2_ref, g2_ref, dinv_ref, b2_ref, out_ref):
    ssum = p2_ref[0] + p2_ref[1] + g2_ref[...]
    out_ref[...] = ssum[:, :2] * dinv_ref[...] + b2_ref[...]


def _tc3(p2, g2, dinv, b2r):
    return pl.pallas_call(
        _tc3_body,
        grid=(NP // _B,),
        in_specs=[
            pl.BlockSpec((2, _B, 16), lambda i: (0, i, 0)),
            pl.BlockSpec((_B, 16), lambda i: (i, 0)),
            pl.BlockSpec((_B, 1), lambda i: (i, 0)),
            pl.BlockSpec((1, 2), lambda i: (0, 0)),
        ],
        out_specs=pl.BlockSpec((_B, 2), lambda i: (i, 0)),
        out_shape=jax.ShapeDtypeStruct((NP, 2), jnp.float32),
    )(p2, g2, dinv, b2r)


def kernel(x, edge_index, W1, b1, W2, b2):
    src = edge_index[0].astype(jnp.int32)
    dst = edge_index[1].astype(jnp.int32)
    pad = jnp.full((EP - E,), N, jnp.int32)
    src_m = jnp.concatenate([src, pad]).reshape(EP // CHUNK, CHUNK)
    dst_m = jnp.concatenate([dst, pad]).reshape(EP // CHUNK, CHUNK)
    x_p = jnp.pad(x, ((0, NP - N), (0, 0)))

    zeros1 = jnp.zeros((NP,), jnp.float32)
    zeros64 = jnp.zeros((NP, 64), jnp.float32)
    zeros16 = jnp.zeros((NP, 16), jnp.float32)
    W2p = jnp.pad(W2, ((0, 0), (0, 16 - 2)))

    degp = _deg_kernel(dst_m, zeros1)
    degp3 = degp.reshape(2, NP, 1)
    g1, dinv = _tc1(x_p, W1, degp3)
    p1 = _agg64(g1, src_m, dst_m, zeros64).reshape(2, NP, 64)
    g2 = _tc2(p1, g1, dinv, b1.reshape(1, 64), W2p)
    p2 = _agg16(g2, src_m, dst_m, zeros16).reshape(2, NP, 16)
    out = _tc3(p2, g2, dinv, b2.reshape(1, 2))
    return out[:N]


# speedup vs baseline: 41.9097x; 2.0313x over previous
import functools

import jax
import jax.numpy as jnp
from jax import lax
from jax.experimental import pallas as pl
from jax.experimental.pallas import tpu as pltpu
from jax.experimental.pallas import tpu_sc as plsc

N = 10000
NP = 10240
E = 320000
NW = 32
CHUNK = 128
CHUNKS = 80
ROWS = NW * CHUNKS
EP = ROWS * CHUNK
CHUNKS_ALL = ROWS // 16
RPT = NP // 16

_mesh = plsc.VectorSubcoreMesh(
    core_axis_name="c", subcore_axis_name="s", num_cores=2, num_subcores=16)


@functools.partial(
    pl.kernel,
    out_type=jax.ShapeDtypeStruct((2 * NP,), jnp.float32),
    mesh=_mesh,
    scratch_types=[
        pltpu.VMEM((CHUNKS, CHUNK), jnp.int32),
        pltpu.VMEM((CHUNK,), jnp.float32),
        pltpu.VMEM((RPT,), jnp.float32),
        pltpu.VMEM_SHARED((NP,), jnp.float32),
        pltpu.SemaphoreType.DMA,
    ],
)
def _deg_kernel(dst_hbm, zeros_hbm, out_hbm, idx_v, ones_v, stage_v, acc_sh, sem):
    c = lax.axis_index("c")
    s = lax.axis_index("s")
    wid = s * 2 + c
    pltpu.sync_copy(zeros_hbm.at[pl.ds(s * RPT, RPT)], stage_v)
    pltpu.sync_copy(stage_v, acc_sh.at[pl.ds(s * RPT, RPT)])
    for i in range(CHUNK // 16):
        ones_v[pl.ds(i * 16, 16)] = jnp.ones((16,), jnp.float32)
    pltpu.sync_copy(dst_hbm.at[pl.ds(wid * CHUNKS, CHUNKS)], idx_v)
    plsc.subcore_barrier()

    NB = 8
    for b in range(NB):
        pltpu.async_copy(ones_v, acc_sh.at[idx_v.at[b]], sem, add=True)

    def body(j, carry):
        pltpu.async_copy(ones_v, acc_sh.at[idx_v.at[j + NB]], sem, add=True)
        pltpu.make_async_copy(ones_v, acc_sh.at[idx_v.at[j]], sem).wait()
        return carry

    lax.fori_loop(0, CHUNKS - NB, body, 0)
    for b in range(NB):
        pltpu.make_async_copy(ones_v, acc_sh.at[idx_v.at[b]], sem).wait()
    plsc.subcore_barrier()
    pltpu.sync_copy(acc_sh.at[pl.ds(s * RPT, RPT)], stage_v)
    pltpu.sync_copy(stage_v, out_hbm.at[pl.ds(c * NP + s * RPT, RPT)])


def _make_agg_halves(D, NSLOT):
    @functools.partial(
        pl.kernel,
        out_type=[jax.ShapeDtypeStruct((NP, D), jnp.float32),
                  jax.ShapeDtypeStruct((NP, D), jnp.float32)],
        mesh=_mesh,
        compiler_params=pltpu.CompilerParams(use_tc_tiling_on_sc=False),
        scratch_types=(
            [pltpu.VMEM((CHUNKS_ALL, CHUNK), jnp.int32),
             pltpu.VMEM((CHUNKS_ALL, CHUNK), jnp.int32),
             pltpu.VMEM((NSLOT, CHUNK, D), jnp.float32),
             pltpu.VMEM_SHARED((NP, D), jnp.float32),
             pltpu.VMEM_SHARED((NP, D), jnp.float32)]
            + [pltpu.SemaphoreType.DMA] * NSLOT
            + [pltpu.SemaphoreType.DMA]
        ),
    )
    def _agg(ga_hbm, gb_hbm, src_hbm, dst_hbm, zeros_hbm, outa_hbm, outb_hbm,
             idxs_v, idxd_v, rows_v, acc_sh, g_sh, *sems):
        gsems = sems[:NSLOT]
        ssem = sems[NSLOT]
        c = lax.axis_index("c")
        s = lax.axis_index("s")

        @pl.when(c == 0)
        def _():
            pltpu.sync_copy(ga_hbm.at[pl.ds(s * RPT, RPT)],
                            g_sh.at[pl.ds(s * RPT, RPT)])

        @pl.when(c == 1)
        def _():
            pltpu.sync_copy(gb_hbm.at[pl.ds(s * RPT, RPT)],
                            g_sh.at[pl.ds(s * RPT, RPT)])

        for i in range(RPT // CHUNK):
            pltpu.sync_copy(zeros_hbm.at[pl.ds(s * RPT + i * CHUNK, CHUNK)],
                            rows_v.at[0])
            pltpu.sync_copy(rows_v.at[0],
                            acc_sh.at[pl.ds(s * RPT + i * CHUNK, CHUNK)])
        pltpu.sync_copy(src_hbm.at[pl.ds(s * CHUNKS_ALL, CHUNKS_ALL)], idxs_v)
        pltpu.sync_copy(dst_hbm.at[pl.ds(s * CHUNKS_ALL, CHUNKS_ALL)], idxd_v)
        plsc.subcore_barrier()

        def gstart(j, b):
            pltpu.async_copy(g_sh.at[idxs_v.at[j]], rows_v.at[b], gsems[b])

        def gwait(j, b):
            pltpu.make_async_copy(g_sh.at[idxs_v.at[j]], rows_v.at[b],
                                  gsems[b]).wait()

        def sstart(j, b):
            pltpu.async_copy(rows_v.at[b], acc_sh.at[idxd_v.at[j]], ssem,
                             add=True)

        def swait(j, b):
            pltpu.make_async_copy(rows_v.at[b], acc_sh.at[idxd_v.at[j]],
                                  ssem).wait()

        nit = CHUNKS_ALL // NSLOT
        for b in range(NSLOT):
            gstart(b, b)

        def body(k, carry):
            j0 = NSLOT * k
            for b in range(NSLOT):
                gwait(j0 + b, b)
                sstart(j0 + b, b)
            for b in range(NSLOT):
                swait(j0 + b, b)

            @pl.when(k < nit - 1)
            def _():
                for b in range(NSLOT):
                    gstart(j0 + NSLOT + b, b)

            return carry

        lax.fori_loop(0, nit, body, 0)
        plsc.subcore_barrier()
        for i in range(RPT // CHUNK):
            pltpu.sync_copy(acc_sh.at[pl.ds(s * RPT + i * CHUNK, CHUNK)],
                            rows_v.at[0])

            @pl.when(c == 0)
            def _():
                pltpu.sync_copy(rows_v.at[0],
                                outa_hbm.at[pl.ds(s * RPT + i * CHUNK, CHUNK)])

            @pl.when(c == 1)
            def _():
                pltpu.sync_copy(rows_v.at[0],
                                outb_hbm.at[pl.ds(s * RPT + i * CHUNK, CHUNK)])

    return _agg


def _make_agg(D, NSLOT):
    @functools.partial(
        pl.kernel,
        out_type=jax.ShapeDtypeStruct((2 * NP, D), jnp.float32),
        mesh=_mesh,
        compiler_params=pltpu.CompilerParams(use_tc_tiling_on_sc=False),
        scratch_types=(
            [pltpu.VMEM((CHUNKS, CHUNK), jnp.int32),
             pltpu.VMEM((CHUNKS, CHUNK), jnp.int32),
             pltpu.VMEM((NSLOT, CHUNK, D), jnp.float32),
             pltpu.VMEM_SHARED((NP, D), jnp.float32),
             pltpu.VMEM_SHARED((NP, D), jnp.float32)]
            + [pltpu.SemaphoreType.DMA] * NSLOT
            + [pltpu.SemaphoreType.DMA]
        ),
    )
    def _agg(g_hbm, src_hbm, dst_hbm, zeros_hbm, out_hbm,
             idxs_v, idxd_v, rows_v, acc_sh, g_sh, *sems):
        gsems = sems[:NSLOT]
        ssem = sems[NSLOT]
        c = lax.axis_index("c")
        s = lax.axis_index("s")
        wid = s * 2 + c
        pltpu.sync_copy(g_hbm.at[pl.ds(s * RPT, RPT)],
                        g_sh.at[pl.ds(s * RPT, RPT)])
        for i in range(RPT // CHUNK):
            pltpu.sync_copy(zeros_hbm.at[pl.ds(s * RPT + i * CHUNK, CHUNK)],
                            rows_v.at[0])
            pltpu.sync_copy(rows_v.at[0],
                            acc_sh.at[pl.ds(s * RPT + i * CHUNK, CHUNK)])
        pltpu.sync_copy(src_hbm.at[pl.ds(wid * CHUNKS, CHUNKS)], idxs_v)
        pltpu.sync_copy(dst_hbm.at[pl.ds(wid * CHUNKS, CHUNKS)], idxd_v)
        plsc.subcore_barrier()

        def gstart(j, b):
            pltpu.async_copy(g_sh.at[idxs_v.at[j]], rows_v.at[b], gsems[b])

        def gwait(j, b):
            pltpu.make_async_copy(g_sh.at[idxs_v.at[j]], rows_v.at[b],
                                  gsems[b]).wait()

        def sstart(j, b):
            pltpu.async_copy(rows_v.at[b], acc_sh.at[idxd_v.at[j]], ssem,
                             add=True)

        def swait(j, b):
            pltpu.make_async_copy(rows_v.at[b], acc_sh.at[idxd_v.at[j]],
                                  ssem).wait()

        nit = CHUNKS // NSLOT
        for b in range(NSLOT):
            gstart(b, b)

        def body(k, carry):
            j0 = NSLOT * k
            for b in range(NSLOT):
                gwait(j0 + b, b)
                sstart(j0 + b, b)
            for b in range(NSLOT):
                swait(j0 + b, b)

            @pl.when(k < nit - 1)
            def _():
                for b in range(NSLOT):
                    gstart(j0 + NSLOT + b, b)

            return carry

        lax.fori_loop(0, nit, body, 0)
        plsc.subcore_barrier()
        for i in range(RPT // CHUNK):
            pltpu.sync_copy(acc_sh.at[pl.ds(s * RPT + i * CHUNK, CHUNK)],
                            rows_v.at[0])
            pltpu.sync_copy(rows_v.at[0],
                            out_hbm.at[pl.ds(c * NP + s * RPT + i * CHUNK,
                                             CHUNK)])

    return _agg


_agg_halves32 = _make_agg_halves(32, 8)
_agg16 = _make_agg(16, 8)


_B = 2048
_BR = _B // CHUNK


def _tc1_body(x_ref, w1_ref, degp_ref, g1a_ref, g1b_ref, dinv_ref):
    deg = degp_ref[0] + degp_ref[1] + 1.0
    dinv = lax.rsqrt(deg)
    dinv_ref[...] = dinv
    h = jnp.dot(x_ref[...], w1_ref[...], preferred_element_type=jnp.float32)
    g1 = h * dinv
    g1a_ref[...] = g1[:, :32]
    g1b_ref[...] = g1[:, 32:]


def _tc1(x_p, W1, degp3):
    return pl.pallas_call(
        _tc1_body,
        grid=(NP // _B,),
        in_specs=[
            pl.BlockSpec((_B, 128), lambda i: (i, 0)),
            pl.BlockSpec((128, 64), lambda i: (0, 0)),
            pl.BlockSpec((2, _B, 1), lambda i: (0, i, 0)),
        ],
        out_specs=[
            pl.BlockSpec((_B, 32), lambda i: (i, 0)),
            pl.BlockSpec((_B, 32), lambda i: (i, 0)),
            pl.BlockSpec((_B, 1), lambda i: (i, 0)),
        ],
        out_shape=[
            jax.ShapeDtypeStruct((NP, 32), jnp.float32),
            jax.ShapeDtypeStruct((NP, 32), jnp.float32),
            jax.ShapeDtypeStruct((NP, 1), jnp.float32),
        ],
    )(x_p, W1, degp3)


def _tc2_body(p1a_ref, p1b_ref, g1a_ref, g1b_ref, dinv_ref, b1_ref, w2_ref,
              g2_ref):
    dinv = dinv_ref[...]
    sa = p1a_ref[...] + g1a_ref[...]
    sb = p1b_ref[...] + g1b_ref[...]
    ssum = jnp.concatenate([sa, sb], axis=1)
    out1 = ssum * dinv + b1_ref[...]
    r = jnp.maximum(out1, 0.0)
    h2 = jnp.dot(r, w2_ref[...], preferred_element_type=jnp.float32)
    g2_ref[...] = h2 * dinv


def _tc2(p1a, p1b, g1a, g1b, dinv, b1r, W2):
    return pl.pallas_call(
        _tc2_body,
        grid=(NP // _B,),
        in_specs=[
            pl.BlockSpec((_B, 32), lambda i: (i, 0)),
            pl.BlockSpec((_B, 32), lambda i: (i, 0)),
            pl.BlockSpec((_B, 32), lambda i: (i, 0)),
            pl.BlockSpec((_B, 32), lambda i: (i, 0)),
            pl.BlockSpec((_B, 1), lambda i: (i, 0)),
            pl.BlockSpec((1, 64), lambda i: (0, 0)),
            pl.BlockSpec((64, 16), lambda i: (0, 0)),
        ],
        out_specs=pl.BlockSpec((_B, 16), lambda i: (i, 0)),
        out_shape=jax.ShapeDtypeStruct((NP, 16), jnp.float32),
    )(p1a, p1b, g1a, g1b, dinv, b1r, W2)


def _tc3_body(p2_ref, g2_ref, dinv_ref, b2_ref, out_ref):
    dinv = dinv_ref[...]
    ssum = p2_ref[0] + p2_ref[1] + g2_ref[...]
    out_ref[...] = ssum[:, :2] * dinv + b2_ref[...]


def _tc3(p2, g2, dinv, b2r):
    return pl.pallas_call(
        _tc3_body,
        grid=(NP // _B,),
        in_specs=[
            pl.BlockSpec((2, _B, 16), lambda i: (0, i, 0)),
            pl.BlockSpec((_B, 16), lambda i: (i, 0)),
            pl.BlockSpec((_B, 1), lambda i: (i, 0)),
            pl.BlockSpec((1, 2), lambda i: (0, 0)),
        ],
        out_specs=pl.BlockSpec((_B, 2), lambda i: (i, 0)),
        out_shape=jax.ShapeDtypeStruct((NP, 2), jnp.float32),
    )(p2, g2, dinv, b2r)


def kernel(x, edge_index, W1, b1, W2, b2):
    ei = edge_index.astype(jnp.int32)
    em = jnp.pad(ei, ((0, 0), (0, EP - E)),
                 constant_values=N).reshape(2, ROWS, CHUNK)
    src_m = em[0]
    dst_m = em[1]
    x_p = jnp.pad(x, ((0, NP - N), (0, 0)))

    zeros1 = jnp.zeros((NP,), jnp.float32)
    zeros32 = jnp.zeros((NP, 32), jnp.float32)
    zeros16 = jnp.zeros((NP, 16), jnp.float32)
    W2p = jnp.pad(W2, ((0, 0), (0, 16 - 2)))

    degp3 = _deg_kernel(dst_m, zeros1).reshape(2, NP, 1)
    g1a, g1b, dinv = _tc1(x_p, W1, degp3)
    p1a, p1b = _agg_halves32(g1a, g1b, src_m, dst_m, zeros32)
    g2 = _tc2(p1a, p1b, g1a, g1b, dinv, b1.reshape(1, 64), W2p)
    p2 = _agg16(g2, src_m, dst_m, zeros16).reshape(2, NP, 16)
    out = _tc3(p2, g2, dinv, b2.reshape(1, 2))
    return out[:N]


# speedup vs baseline: 43.0820x; 1.0280x over previous
import functools

import jax
import jax.numpy as jnp
from jax import lax
from jax.experimental import pallas as pl
from jax.experimental.pallas import tpu as pltpu
from jax.experimental.pallas import tpu_sc as plsc

N = 10000
NP = 10240
E = 320000
NW = 32
CHUNK = 128
CHUNKS = 80
ROWS = NW * CHUNKS
EP = ROWS * CHUNK
CHUNKS_ALL = ROWS // 16
RPT = NP // 16

_mesh = plsc.VectorSubcoreMesh(
    core_axis_name="c", subcore_axis_name="s", num_cores=2, num_subcores=16)


@functools.partial(
    pl.kernel,
    out_type=jax.ShapeDtypeStruct((2 * NP,), jnp.float32),
    mesh=_mesh,
    scratch_types=[
        pltpu.VMEM((CHUNKS, CHUNK), jnp.int32),
        pltpu.VMEM((CHUNK,), jnp.float32),
        pltpu.VMEM((RPT,), jnp.float32),
        pltpu.VMEM_SHARED((NP,), jnp.float32),
        pltpu.SemaphoreType.DMA,
    ],
)
def _deg_kernel(dst_hbm, zeros_hbm, out_hbm, idx_v, ones_v, stage_v, acc_sh, sem):
    c = lax.axis_index("c")
    s = lax.axis_index("s")
    wid = s * 2 + c
    pltpu.sync_copy(zeros_hbm.at[pl.ds(s * RPT, RPT)], stage_v)
    pltpu.sync_copy(stage_v, acc_sh.at[pl.ds(s * RPT, RPT)])
    for i in range(CHUNK // 16):
        ones_v[pl.ds(i * 16, 16)] = jnp.ones((16,), jnp.float32)
    pltpu.sync_copy(dst_hbm.at[pl.ds(wid * CHUNKS, CHUNKS)], idx_v)
    plsc.subcore_barrier()

    NB = 8
    for b in range(NB):
        pltpu.async_copy(ones_v, acc_sh.at[idx_v.at[b]], sem, add=True)

    def body(j, carry):
        pltpu.async_copy(ones_v, acc_sh.at[idx_v.at[j + NB]], sem, add=True)
        pltpu.make_async_copy(ones_v, acc_sh.at[idx_v.at[j]], sem).wait()
        return carry

    lax.fori_loop(0, CHUNKS - NB, body, 0)
    for b in range(NB):
        pltpu.make_async_copy(ones_v, acc_sh.at[idx_v.at[b]], sem).wait()
    plsc.subcore_barrier()
    pltpu.sync_copy(acc_sh.at[pl.ds(s * RPT, RPT)], stage_v)
    pltpu.sync_copy(stage_v, out_hbm.at[pl.ds(c * NP + s * RPT, RPT)])


def _make_agg_halves(D, NSLOT):
    @functools.partial(
        pl.kernel,
        out_type=[jax.ShapeDtypeStruct((NP, D), jnp.float32),
                  jax.ShapeDtypeStruct((NP, D), jnp.float32)],
        mesh=_mesh,
        compiler_params=pltpu.CompilerParams(use_tc_tiling_on_sc=False),
        scratch_types=(
            [pltpu.VMEM((CHUNKS_ALL, CHUNK), jnp.int32),
             pltpu.VMEM((CHUNKS_ALL, CHUNK), jnp.int32),
             pltpu.VMEM((NSLOT, CHUNK, D), jnp.float32),
             pltpu.VMEM_SHARED((NP, D), jnp.float32),
             pltpu.VMEM_SHARED((NP, D), jnp.float32)]
            + [pltpu.SemaphoreType.DMA] * NSLOT
            + [pltpu.SemaphoreType.DMA]
        ),
    )
    def _agg(ga_hbm, gb_hbm, src_hbm, dst_hbm, zeros_hbm, outa_hbm, outb_hbm,
             idxs_v, idxd_v, rows_v, acc_sh, g_sh, *sems):
        gsems = sems[:NSLOT]
        ssem = sems[NSLOT]
        c = lax.axis_index("c")
        s = lax.axis_index("s")

        @pl.when(c == 0)
        def _():
            pltpu.sync_copy(ga_hbm.at[pl.ds(s * RPT, RPT)],
                            g_sh.at[pl.ds(s * RPT, RPT)])

        @pl.when(c == 1)
        def _():
            pltpu.sync_copy(gb_hbm.at[pl.ds(s * RPT, RPT)],
                            g_sh.at[pl.ds(s * RPT, RPT)])

        for i in range(RPT // CHUNK):
            pltpu.sync_copy(zeros_hbm.at[pl.ds(s * RPT + i * CHUNK, CHUNK)],
                            rows_v.at[0])
            pltpu.sync_copy(rows_v.at[0],
                            acc_sh.at[pl.ds(s * RPT + i * CHUNK, CHUNK)])
        pltpu.sync_copy(src_hbm.at[pl.ds(s * CHUNKS_ALL, CHUNKS_ALL)], idxs_v)
        pltpu.sync_copy(dst_hbm.at[pl.ds(s * CHUNKS_ALL, CHUNKS_ALL)], idxd_v)
        plsc.subcore_barrier()

        def gstart(j, b):
            pltpu.async_copy(g_sh.at[idxs_v.at[j]], rows_v.at[b], gsems[b])

        def gwait(j, b):
            pltpu.make_async_copy(g_sh.at[idxs_v.at[j]], rows_v.at[b],
                                  gsems[b]).wait()

        def sstart(j, b):
            pltpu.async_copy(rows_v.at[b], acc_sh.at[idxd_v.at[j]], ssem,
                             add=True)

        def swait(j, b):
            pltpu.make_async_copy(rows_v.at[b], acc_sh.at[idxd_v.at[j]],
                                  ssem).wait()

        nit = CHUNKS_ALL // NSLOT
        for b in range(NSLOT):
            gstart(b, b)

        def body(k, carry):
            j0 = NSLOT * k
            for b in range(NSLOT):
                gwait(j0 + b, b)
                sstart(j0 + b, b)
            for b in range(NSLOT):
                swait(j0 + b, b)

            @pl.when(k < nit - 1)
            def _():
                for b in range(NSLOT):
                    gstart(j0 + NSLOT + b, b)

            return carry

        lax.fori_loop(0, nit, body, 0)
        plsc.subcore_barrier()
        for i in range(RPT // CHUNK):
            pltpu.sync_copy(acc_sh.at[pl.ds(s * RPT + i * CHUNK, CHUNK)],
                            rows_v.at[0])

            @pl.when(c == 0)
            def _():
                pltpu.sync_copy(rows_v.at[0],
                                outa_hbm.at[pl.ds(s * RPT + i * CHUNK, CHUNK)])

            @pl.when(c == 1)
            def _():
                pltpu.sync_copy(rows_v.at[0],
                                outb_hbm.at[pl.ds(s * RPT + i * CHUNK, CHUNK)])

    return _agg


def _make_agg(D, NSLOT):
    @functools.partial(
        pl.kernel,
        out_type=jax.ShapeDtypeStruct((2 * NP, D), jnp.float32),
        mesh=_mesh,
        compiler_params=pltpu.CompilerParams(use_tc_tiling_on_sc=False),
        scratch_types=(
            [pltpu.VMEM((CHUNKS, CHUNK), jnp.int32),
             pltpu.VMEM((CHUNKS, CHUNK), jnp.int32),
             pltpu.VMEM((NSLOT, CHUNK, D), jnp.float32),
             pltpu.VMEM_SHARED((NP, D), jnp.float32),
             pltpu.VMEM_SHARED((NP, D), jnp.float32)]
            + [pltpu.SemaphoreType.DMA] * NSLOT
            + [pltpu.SemaphoreType.DMA]
        ),
    )
    def _agg(g_hbm, src_hbm, dst_hbm, zeros_hbm, out_hbm,
             idxs_v, idxd_v, rows_v, acc_sh, g_sh, *sems):
        gsems = sems[:NSLOT]
        ssem = sems[NSLOT]
        c = lax.axis_index("c")
        s = lax.axis_index("s")
        wid = s * 2 + c
        pltpu.sync_copy(g_hbm.at[pl.ds(s * RPT, RPT)],
                        g_sh.at[pl.ds(s * RPT, RPT)])
        for i in range(RPT // CHUNK):
            pltpu.sync_copy(zeros_hbm.at[pl.ds(s * RPT + i * CHUNK, CHUNK)],
                            rows_v.at[0])
            pltpu.sync_copy(rows_v.at[0],
                            acc_sh.at[pl.ds(s * RPT + i * CHUNK, CHUNK)])
        pltpu.sync_copy(src_hbm.at[pl.ds(wid * CHUNKS, CHUNKS)], idxs_v)
        pltpu.sync_copy(dst_hbm.at[pl.ds(wid * CHUNKS, CHUNKS)], idxd_v)
        plsc.subcore_barrier()

        def gstart(j, b):
            pltpu.async_copy(g_sh.at[idxs_v.at[j]], rows_v.at[b], gsems[b])

        def gwait(j, b):
            pltpu.make_async_copy(g_sh.at[idxs_v.at[j]], rows_v.at[b],
                                  gsems[b]).wait()

        def sstart(j, b):
            pltpu.async_copy(rows_v.at[b], acc_sh.at[idxd_v.at[j]], ssem,
                             add=True)

        def swait(j, b):
            pltpu.make_async_copy(rows_v.at[b], acc_sh.at[idxd_v.at[j]],
                                  ssem).wait()

        nit = CHUNKS // NSLOT
        for b in range(NSLOT):
            gstart(b, b)

        def body(k, carry):
            j0 = NSLOT * k
            for b in range(NSLOT):
                gwait(j0 + b, b)
                sstart(j0 + b, b)
            for b in range(NSLOT):
                swait(j0 + b, b)

            @pl.when(k < nit - 1)
            def _():
                for b in range(NSLOT):
                    gstart(j0 + NSLOT + b, b)

            return carry

        lax.fori_loop(0, nit, body, 0)
        plsc.subcore_barrier()
        for i in range(RPT // CHUNK):
            pltpu.sync_copy(acc_sh.at[pl.ds(s * RPT + i * CHUNK, CHUNK)],
                            rows_v.at[0])
            pltpu.sync_copy(rows_v.at[0],
                            out_hbm.at[pl.ds(c * NP + s * RPT + i * CHUNK,
                                             CHUNK)])

    return _agg


_agg_halves32 = _make_agg_halves(32, 8)
_agg8 = _make_agg(8, 8)


_B = 2048
_BR = _B // CHUNK


def _tc1_body(x_ref, w1_ref, degp_ref, g1a_ref, g1b_ref, dinv_ref):
    deg = degp_ref[0] + degp_ref[1] + 1.0
    dinv = lax.rsqrt(deg)
    dinv_ref[...] = dinv
    h = jnp.dot(x_ref[...], w1_ref[...], preferred_element_type=jnp.float32)
    g1 = h * dinv
    g1a_ref[...] = g1[:, :32]
    g1b_ref[...] = g1[:, 32:]


def _tc1(x_p, W1, degp3):
    return pl.pallas_call(
        _tc1_body,
        grid=(NP // _B,),
        in_specs=[
            pl.BlockSpec((_B, 128), lambda i: (i, 0)),
            pl.BlockSpec((128, 64), lambda i: (0, 0)),
            pl.BlockSpec((2, _B, 1), lambda i: (0, i, 0)),
        ],
        out_specs=[
            pl.BlockSpec((_B, 32), lambda i: (i, 0)),
            pl.BlockSpec((_B, 32), lambda i: (i, 0)),
            pl.BlockSpec((_B, 1), lambda i: (i, 0)),
        ],
        out_shape=[
            jax.ShapeDtypeStruct((NP, 32), jnp.float32),
            jax.ShapeDtypeStruct((NP, 32), jnp.float32),
            jax.ShapeDtypeStruct((NP, 1), jnp.float32),
        ],
    )(x_p, W1, degp3)


def _tc2_body(p1a_ref, p1b_ref, g1a_ref, g1b_ref, dinv_ref, b1_ref, w2_ref,
              g2_ref):
    dinv = dinv_ref[...]
    sa = p1a_ref[...] + g1a_ref[...]
    sb = p1b_ref[...] + g1b_ref[...]
    ssum = jnp.concatenate([sa, sb], axis=1)
    out1 = ssum * dinv + b1_ref[...]
    r = jnp.maximum(out1, 0.0)
    h2 = jnp.dot(r, w2_ref[...], preferred_element_type=jnp.float32)
    g2_ref[...] = h2 * dinv


def _tc2(p1a, p1b, g1a, g1b, dinv, b1r, W2):
    return pl.pallas_call(
        _tc2_body,
        grid=(NP // _B,),
        in_specs=[
            pl.BlockSpec((_B, 32), lambda i: (i, 0)),
            pl.BlockSpec((_B, 32), lambda i: (i, 0)),
            pl.BlockSpec((_B, 32), lambda i: (i, 0)),
            pl.BlockSpec((_B, 32), lambda i: (i, 0)),
            pl.BlockSpec((_B, 1), lambda i: (i, 0)),
            pl.BlockSpec((1, 64), lambda i: (0, 0)),
            pl.BlockSpec((64, 8), lambda i: (0, 0)),
        ],
        out_specs=pl.BlockSpec((_B, 8), lambda i: (i, 0)),
        out_shape=jax.ShapeDtypeStruct((NP, 8), jnp.float32),
    )(p1a, p1b, g1a, g1b, dinv, b1r, W2)


def _tc3_body(p2_ref, g2_ref, dinv_ref, b2_ref, out_ref):
    dinv = dinv_ref[...]
    ssum = p2_ref[0] + p2_ref[1] + g2_ref[...]
    out_ref[...] = ssum[:, :2] * dinv + b2_ref[...]


def _tc3(p2, g2, dinv, b2r):
    return pl.pallas_call(
        _tc3_body,
        grid=(NP // _B,),
        in_specs=[
            pl.BlockSpec((2, _B, 8), lambda i: (0, i, 0)),
            pl.BlockSpec((_B, 8), lambda i: (i, 0)),
            pl.BlockSpec((_B, 1), lambda i: (i, 0)),
            pl.BlockSpec((1, 2), lambda i: (0, 0)),
        ],
        out_specs=pl.BlockSpec((_B, 2), lambda i: (i, 0)),
        out_shape=jax.ShapeDtypeStruct((NP, 2), jnp.float32),
    )(p2, g2, dinv, b2r)


def kernel(x, edge_index, W1, b1, W2, b2):
    ei = edge_index.astype(jnp.int32)
    em = jnp.pad(ei, ((0, 0), (0, EP - E)),
                 constant_values=N).reshape(2, ROWS, CHUNK)
    src_m = em[0]
    dst_m = em[1]
    x_p = jnp.pad(x, ((0, NP - N), (0, 0)))

    zeros1 = jnp.zeros((NP,), jnp.float32)
    zeros32 = jnp.zeros((NP, 32), jnp.float32)
    zeros8 = jnp.zeros((NP, 8), jnp.float32)
    W2p = jnp.pad(W2, ((0, 0), (0, 8 - 2)))

    degp3 = _deg_kernel(dst_m, zeros1).reshape(2, NP, 1)
    g1a, g1b, dinv = _tc1(x_p, W1, degp3)
    p1a, p1b = _agg_halves32(g1a, g1b, src_m, dst_m, zeros32)
    g2 = _tc2(p1a, p1b, g1a, g1b, dinv, b1.reshape(1, 64), W2p)
    p2 = _agg8(g2, src_m, dst_m, zeros8).reshape(2, NP, 8)
    out = _tc3(p2, g2, dinv, b2.reshape(1, 2))
    return out[:N]


# speedup vs baseline: 51.6350x; 1.1985x over previous
import functools

import jax
import jax.numpy as jnp
from jax import lax
from jax.experimental import pallas as pl
from jax.experimental.pallas import tpu as pltpu
from jax.experimental.pallas import tpu_sc as plsc

N = 10000
NP = 10240
E = 320000
NW = 32
CHUNK = 128
CHUNKS = 80
ROWS = NW * CHUNKS
EP = ROWS * CHUNK
CHUNKS_ALL = ROWS // 16
RPT = NP // 16

_mesh = plsc.VectorSubcoreMesh(
    core_axis_name="c", subcore_axis_name="s", num_cores=2, num_subcores=16)


@functools.partial(
    pl.kernel,
    out_type=jax.ShapeDtypeStruct((2 * NP,), jnp.float32),
    mesh=_mesh,
    scratch_types=[
        pltpu.VMEM((CHUNKS, CHUNK), jnp.int32),
        pltpu.VMEM((CHUNK,), jnp.float32),
        pltpu.VMEM((RPT,), jnp.float32),
        pltpu.VMEM_SHARED((NP,), jnp.float32),
        pltpu.SemaphoreType.DMA,
    ],
)
def _deg_kernel(dst_hbm, zeros_hbm, out_hbm, idx_v, ones_v, stage_v, acc_sh, sem):
    c = lax.axis_index("c")
    s = lax.axis_index("s")
    wid = s * 2 + c
    pltpu.sync_copy(zeros_hbm.at[pl.ds(s * RPT, RPT)], stage_v)
    pltpu.sync_copy(stage_v, acc_sh.at[pl.ds(s * RPT, RPT)])
    for i in range(CHUNK // 16):
        ones_v[pl.ds(i * 16, 16)] = jnp.ones((16,), jnp.float32)
    pltpu.sync_copy(dst_hbm.at[pl.ds(wid * CHUNKS, CHUNKS)], idx_v)
    plsc.subcore_barrier()

    NB = 8
    for b in range(NB):
        pltpu.async_copy(ones_v, acc_sh.at[idx_v.at[b]], sem, add=True)

    def body(j, carry):
        pltpu.async_copy(ones_v, acc_sh.at[idx_v.at[j + NB]], sem, add=True)
        pltpu.make_async_copy(ones_v, acc_sh.at[idx_v.at[j]], sem).wait()
        return carry

    lax.fori_loop(0, CHUNKS - NB, body, 0)
    for b in range(NB):
        pltpu.make_async_copy(ones_v, acc_sh.at[idx_v.at[b]], sem).wait()
    plsc.subcore_barrier()
    pltpu.sync_copy(acc_sh.at[pl.ds(s * RPT, RPT)], stage_v)
    pltpu.sync_copy(stage_v, out_hbm.at[pl.ds(c * NP + s * RPT, RPT)])


def _make_agg_halves(D, NSLOT):
    @functools.partial(
        pl.kernel,
        out_type=[jax.ShapeDtypeStruct((NP, D), jnp.bfloat16),
                  jax.ShapeDtypeStruct((NP, D), jnp.bfloat16)],
        mesh=_mesh,
        compiler_params=pltpu.CompilerParams(use_tc_tiling_on_sc=False),
        scratch_types=(
            [pltpu.VMEM((CHUNKS_ALL, CHUNK), jnp.int32),
             pltpu.VMEM((CHUNKS_ALL, CHUNK), jnp.int32),
             pltpu.VMEM((NSLOT, CHUNK, D), jnp.bfloat16),
             pltpu.VMEM_SHARED((NP, D), jnp.bfloat16),
             pltpu.VMEM_SHARED((NP, D), jnp.bfloat16)]
            + [pltpu.SemaphoreType.DMA] * NSLOT
            + [pltpu.SemaphoreType.DMA]
        ),
    )
    def _agg(ga_hbm, gb_hbm, src_hbm, dst_hbm, zeros_hbm, outa_hbm, outb_hbm,
             idxs_v, idxd_v, rows_v, acc_sh, g_sh, *sems):
        gsems = sems[:NSLOT]
        ssem = sems[NSLOT]
        c = lax.axis_index("c")
        s = lax.axis_index("s")

        @pl.when(c == 0)
        def _():
            pltpu.sync_copy(ga_hbm.at[pl.ds(s * RPT, RPT)],
                            g_sh.at[pl.ds(s * RPT, RPT)])

        @pl.when(c == 1)
        def _():
            pltpu.sync_copy(gb_hbm.at[pl.ds(s * RPT, RPT)],
                            g_sh.at[pl.ds(s * RPT, RPT)])

        for i in range(RPT // CHUNK):
            pltpu.sync_copy(zeros_hbm.at[pl.ds(s * RPT + i * CHUNK, CHUNK)],
                            rows_v.at[0])
            pltpu.sync_copy(rows_v.at[0],
                            acc_sh.at[pl.ds(s * RPT + i * CHUNK, CHUNK)])
        pltpu.sync_copy(src_hbm.at[pl.ds(s * CHUNKS_ALL, CHUNKS_ALL)], idxs_v)
        pltpu.sync_copy(dst_hbm.at[pl.ds(s * CHUNKS_ALL, CHUNKS_ALL)], idxd_v)
        plsc.subcore_barrier()

        def gstart(j, b):
            pltpu.async_copy(g_sh.at[idxs_v.at[j]], rows_v.at[b], gsems[b])

        def gwait(j, b):
            pltpu.make_async_copy(g_sh.at[idxs_v.at[j]], rows_v.at[b],
                                  gsems[b]).wait()

        def sstart(j, b):
            pltpu.async_copy(rows_v.at[b], acc_sh.at[idxd_v.at[j]], ssem,
                             add=True)

        def swait(j, b):
            pltpu.make_async_copy(rows_v.at[b], acc_sh.at[idxd_v.at[j]],
                                  ssem).wait()

        nit = CHUNKS_ALL // NSLOT
        for b in range(NSLOT):
            gstart(b, b)

        def body(k, carry):
            j0 = NSLOT * k
            for b in range(NSLOT):
                gwait(j0 + b, b)
                sstart(j0 + b, b)
            for b in range(NSLOT):
                swait(j0 + b, b)

            @pl.when(k < nit - 1)
            def _():
                for b in range(NSLOT):
                    gstart(j0 + NSLOT + b, b)

            return carry

        lax.fori_loop(0, nit, body, 0)
        plsc.subcore_barrier()
        for i in range(RPT // CHUNK):
            pltpu.sync_copy(acc_sh.at[pl.ds(s * RPT + i * CHUNK, CHUNK)],
                            rows_v.at[0])

            @pl.when(c == 0)
            def _():
                pltpu.sync_copy(rows_v.at[0],
                                outa_hbm.at[pl.ds(s * RPT + i * CHUNK, CHUNK)])

            @pl.when(c == 1)
            def _():
                pltpu.sync_copy(rows_v.at[0],
                                outb_hbm.at[pl.ds(s * RPT + i * CHUNK, CHUNK)])

    return _agg


def _make_agg(D, NSLOT):
    @functools.partial(
        pl.kernel,
        out_type=jax.ShapeDtypeStruct((2 * NP, D), jnp.float32),
        mesh=_mesh,
        compiler_params=pltpu.CompilerParams(use_tc_tiling_on_sc=False),
        scratch_types=(
            [pltpu.VMEM((CHUNKS, CHUNK), jnp.int32),
             pltpu.VMEM((CHUNKS, CHUNK), jnp.int32),
             pltpu.VMEM((NSLOT, CHUNK, D), jnp.float32),
             pltpu.VMEM_SHARED((NP, D), jnp.float32),
             pltpu.VMEM_SHARED((NP, D), jnp.float32)]
            + [pltpu.SemaphoreType.DMA] * NSLOT
            + [pltpu.SemaphoreType.DMA]
        ),
    )
    def _agg(g_hbm, src_hbm, dst_hbm, zeros_hbm, out_hbm,
             idxs_v, idxd_v, rows_v, acc_sh, g_sh, *sems):
        gsems = sems[:NSLOT]
        ssem = sems[NSLOT]
        c = lax.axis_index("c")
        s = lax.axis_index("s")
        wid = s * 2 + c
        pltpu.sync_copy(g_hbm.at[pl.ds(s * RPT, RPT)],
                        g_sh.at[pl.ds(s * RPT, RPT)])
        for i in range(RPT // CHUNK):
            pltpu.sync_copy(zeros_hbm.at[pl.ds(s * RPT + i * CHUNK, CHUNK)],
                            rows_v.at[0])
            pltpu.sync_copy(rows_v.at[0],
                            acc_sh.at[pl.ds(s * RPT + i * CHUNK, CHUNK)])
        pltpu.sync_copy(src_hbm.at[pl.ds(wid * CHUNKS, CHUNKS)], idxs_v)
        pltpu.sync_copy(dst_hbm.at[pl.ds(wid * CHUNKS, CHUNKS)], idxd_v)
        plsc.subcore_barrier()

        def gstart(j, b):
            pltpu.async_copy(g_sh.at[idxs_v.at[j]], rows_v.at[b], gsems[b])

        def gwait(j, b):
            pltpu.make_async_copy(g_sh.at[idxs_v.at[j]], rows_v.at[b],
                                  gsems[b]).wait()

        def sstart(j, b):
            pltpu.async_copy(rows_v.at[b], acc_sh.at[idxd_v.at[j]], ssem,
                             add=True)

        def swait(j, b):
            pltpu.make_async_copy(rows_v.at[b], acc_sh.at[idxd_v.at[j]],
                                  ssem).wait()

        nit = CHUNKS // NSLOT
        for b in range(NSLOT):
            gstart(b, b)

        def body(k, carry):
            j0 = NSLOT * k
            for b in range(NSLOT):
                gwait(j0 + b, b)
                sstart(j0 + b, b)
            for b in range(NSLOT):
                swait(j0 + b, b)

            @pl.when(k < nit - 1)
            def _():
                for b in range(NSLOT):
                    gstart(j0 + NSLOT + b, b)

            return carry

        lax.fori_loop(0, nit, body, 0)
        plsc.subcore_barrier()
        for i in range(RPT // CHUNK):
            pltpu.sync_copy(acc_sh.at[pl.ds(s * RPT + i * CHUNK, CHUNK)],
                            rows_v.at[0])
            pltpu.sync_copy(rows_v.at[0],
                            out_hbm.at[pl.ds(c * NP + s * RPT + i * CHUNK,
                                             CHUNK)])

    return _agg


_agg_halves32 = _make_agg_halves(32, 8)
_agg8 = _make_agg(8, 8)


_B = 2048
_BR = _B // CHUNK


def _tc1_body(x_ref, w1_ref, degp_ref, g1a_ref, g1b_ref, dinv_ref):
    deg = degp_ref[0] + degp_ref[1] + 1.0
    dinv = lax.rsqrt(deg)
    dinv_ref[...] = dinv
    h = jnp.dot(x_ref[...], w1_ref[...], preferred_element_type=jnp.float32)
    g1 = (h * dinv).astype(jnp.bfloat16)
    g1a_ref[...] = g1[:, :32]
    g1b_ref[...] = g1[:, 32:]


def _tc1(x_p, W1, degp3):
    return pl.pallas_call(
        _tc1_body,
        grid=(NP // _B,),
        in_specs=[
            pl.BlockSpec((_B, 128), lambda i: (i, 0)),
            pl.BlockSpec((128, 64), lambda i: (0, 0)),
            pl.BlockSpec((2, _B, 1), lambda i: (0, i, 0)),
        ],
        out_specs=[
            pl.BlockSpec((_B, 32), lambda i: (i, 0)),
            pl.BlockSpec((_B, 32), lambda i: (i, 0)),
            pl.BlockSpec((_B, 1), lambda i: (i, 0)),
        ],
        out_shape=[
            jax.ShapeDtypeStruct((NP, 32), jnp.bfloat16),
            jax.ShapeDtypeStruct((NP, 32), jnp.bfloat16),
            jax.ShapeDtypeStruct((NP, 1), jnp.float32),
        ],
    )(x_p, W1, degp3)


def _tc2_body(p1a_ref, p1b_ref, g1a_ref, g1b_ref, dinv_ref, b1_ref, w2_ref,
              g2_ref):
    dinv = dinv_ref[...]
    sa = (p1a_ref[...] + g1a_ref[...]).astype(jnp.float32)
    sb = (p1b_ref[...] + g1b_ref[...]).astype(jnp.float32)
    ssum = jnp.concatenate([sa, sb], axis=1)
    out1 = ssum * dinv + b1_ref[...]
    r = jnp.maximum(out1, 0.0)
    h2 = jnp.dot(r, w2_ref[...], preferred_element_type=jnp.float32)
    g2_ref[...] = h2 * dinv


def _tc2(p1a, p1b, g1a, g1b, dinv, b1r, W2):
    return pl.pallas_call(
        _tc2_body,
        grid=(NP // _B,),
        in_specs=[
            pl.BlockSpec((_B, 32), lambda i: (i, 0)),
            pl.BlockSpec((_B, 32), lambda i: (i, 0)),
            pl.BlockSpec((_B, 32), lambda i: (i, 0)),
            pl.BlockSpec((_B, 32), lambda i: (i, 0)),
            pl.BlockSpec((_B, 1), lambda i: (i, 0)),
            pl.BlockSpec((1, 64), lambda i: (0, 0)),
            pl.BlockSpec((64, 8), lambda i: (0, 0)),
        ],
        out_specs=pl.BlockSpec((_B, 8), lambda i: (i, 0)),
        out_shape=jax.ShapeDtypeStruct((NP, 8), jnp.float32),
    )(p1a, p1b, g1a, g1b, dinv, b1r, W2)


def _tc3_body(p2_ref, g2_ref, dinv_ref, b2_ref, out_ref):
    dinv = dinv_ref[...]
    ssum = p2_ref[0] + p2_ref[1] + g2_ref[...]
    out_ref[...] = ssum[:, :2] * dinv + b2_ref[...]


def _tc3(p2, g2, dinv, b2r):
    return pl.pallas_call(
        _tc3_body,
        grid=(NP // _B,),
        in_specs=[
            pl.BlockSpec((2, _B, 8), lambda i: (0, i, 0)),
            pl.BlockSpec((_B, 8), lambda i: (i, 0)),
            pl.BlockSpec((_B, 1), lambda i: (i, 0)),
            pl.BlockSpec((1, 2), lambda i: (0, 0)),
        ],
        out_specs=pl.BlockSpec((_B, 2), lambda i: (i, 0)),
        out_shape=jax.ShapeDtypeStruct((NP, 2), jnp.float32),
    )(p2, g2, dinv, b2r)


def kernel(x, edge_index, W1, b1, W2, b2):
    ei = edge_index.astype(jnp.int32)
    em = jnp.pad(ei, ((0, 0), (0, EP - E)),
                 constant_values=N).reshape(2, ROWS, CHUNK)
    src_m = em[0]
    dst_m = em[1]
    x_p = jnp.pad(x, ((0, NP - N), (0, 0)))

    zeros1 = jnp.zeros((NP,), jnp.float32)
    zeros32 = jnp.zeros((NP, 32), jnp.bfloat16)
    zeros8 = jnp.zeros((NP, 8), jnp.float32)
    W2p = jnp.pad(W2, ((0, 0), (0, 8 - 2)))

    degp3 = _deg_kernel(dst_m, zeros1).reshape(2, NP, 1)
    g1a, g1b, dinv = _tc1(x_p, W1, degp3)
    p1a, p1b = _agg_halves32(g1a, g1b, src_m, dst_m, zeros32)
    g2 = _tc2(p1a, p1b, g1a, g1b, dinv, b1.reshape(1, 64), W2p)
    p2 = _agg8(g2, src_m, dst_m, zeros8).reshape(2, NP, 8)
    out = _tc3(p2, g2, dinv, b2.reshape(1, 2))
    return out[:N]


# speedup vs baseline: 52.1862x; 1.0107x over previous
import functools

import jax
import jax.numpy as jnp
from jax import lax
from jax.experimental import pallas as pl
from jax.experimental.pallas import tpu as pltpu
from jax.experimental.pallas import tpu_sc as plsc

N = 10000
NP = 10240
E = 320000
NW = 32
CHUNK = 128
CHUNKS = 80
ROWS = NW * CHUNKS
EP = ROWS * CHUNK
CHUNKS_ALL = ROWS // 16
RPT = NP // 16

_mesh = plsc.VectorSubcoreMesh(
    core_axis_name="c", subcore_axis_name="s", num_cores=2, num_subcores=16)


@functools.partial(
    pl.kernel,
    out_type=jax.ShapeDtypeStruct((2 * NP,), jnp.float32),
    mesh=_mesh,
    scratch_types=[
        pltpu.VMEM((CHUNKS, CHUNK), jnp.int32),
        pltpu.VMEM((CHUNK,), jnp.float32),
        pltpu.VMEM((RPT,), jnp.float32),
        pltpu.VMEM_SHARED((NP,), jnp.float32),
        pltpu.SemaphoreType.DMA,
    ],
)
def _deg_kernel(dst_hbm, zeros_hbm, out_hbm, idx_v, ones_v, stage_v, acc_sh, sem):
    c = lax.axis_index("c")
    s = lax.axis_index("s")
    wid = s * 2 + c
    pltpu.sync_copy(zeros_hbm.at[pl.ds(s * RPT, RPT)], stage_v)
    pltpu.sync_copy(stage_v, acc_sh.at[pl.ds(s * RPT, RPT)])
    for i in range(CHUNK // 16):
        ones_v[pl.ds(i * 16, 16)] = jnp.ones((16,), jnp.float32)
    pltpu.sync_copy(dst_hbm.at[pl.ds(wid * CHUNKS, CHUNKS)], idx_v)
    plsc.subcore_barrier()

    NB = 8
    for b in range(NB):
        pltpu.async_copy(ones_v, acc_sh.at[idx_v.at[b]], sem, add=True)

    def body(j, carry):
        pltpu.async_copy(ones_v, acc_sh.at[idx_v.at[j + NB]], sem, add=True)
        pltpu.make_async_copy(ones_v, acc_sh.at[idx_v.at[j]], sem).wait()
        return carry

    lax.fori_loop(0, CHUNKS - NB, body, 0)
    for b in range(NB):
        pltpu.make_async_copy(ones_v, acc_sh.at[idx_v.at[b]], sem).wait()
    plsc.subcore_barrier()
    pltpu.sync_copy(acc_sh.at[pl.ds(s * RPT, RPT)], stage_v)
    pltpu.sync_copy(stage_v, out_hbm.at[pl.ds(c * NP + s * RPT, RPT)])


def _make_agg_halves(D, NSLOT):
    @functools.partial(
        pl.kernel,
        out_type=[jax.ShapeDtypeStruct((NP, D), jnp.bfloat16),
                  jax.ShapeDtypeStruct((NP, D), jnp.bfloat16)],
        mesh=_mesh,
        compiler_params=pltpu.CompilerParams(use_tc_tiling_on_sc=False),
        scratch_types=(
            [pltpu.VMEM((CHUNKS_ALL, CHUNK), jnp.int32),
             pltpu.VMEM((CHUNKS_ALL, CHUNK), jnp.int32),
             pltpu.VMEM((NSLOT, CHUNK, D), jnp.bfloat16),
             pltpu.VMEM_SHARED((NP, D), jnp.bfloat16),
             pltpu.VMEM_SHARED((NP, D), jnp.bfloat16)]
            + [pltpu.SemaphoreType.DMA] * NSLOT
            + [pltpu.SemaphoreType.DMA]
        ),
    )
    def _agg(ga_hbm, gb_hbm, src_hbm, dst_hbm, zeros_hbm, outa_hbm, outb_hbm,
             idxs_v, idxd_v, rows_v, acc_sh, g_sh, *sems):
        gsems = sems[:NSLOT]
        ssem = sems[NSLOT]
        c = lax.axis_index("c")
        s = lax.axis_index("s")

        @pl.when(c == 0)
        def _():
            pltpu.sync_copy(ga_hbm.at[pl.ds(s * RPT, RPT)],
                            g_sh.at[pl.ds(s * RPT, RPT)])

        @pl.when(c == 1)
        def _():
            pltpu.sync_copy(gb_hbm.at[pl.ds(s * RPT, RPT)],
                            g_sh.at[pl.ds(s * RPT, RPT)])

        for i in range(RPT // CHUNK):
            pltpu.sync_copy(zeros_hbm.at[pl.ds(s * RPT + i * CHUNK, CHUNK)],
                            rows_v.at[0])
            pltpu.sync_copy(rows_v.at[0],
                            acc_sh.at[pl.ds(s * RPT + i * CHUNK, CHUNK)])
        pltpu.sync_copy(src_hbm.at[pl.ds(s * CHUNKS_ALL, CHUNKS_ALL)], idxs_v)
        pltpu.sync_copy(dst_hbm.at[pl.ds(s * CHUNKS_ALL, CHUNKS_ALL)], idxd_v)
        plsc.subcore_barrier()

        def gstart(j, b):
            pltpu.async_copy(g_sh.at[idxs_v.at[j]], rows_v.at[b], gsems[b])

        def gwait(j, b):
            pltpu.make_async_copy(g_sh.at[idxs_v.at[j]], rows_v.at[b],
                                  gsems[b]).wait()

        def sstart(j, b):
            pltpu.async_copy(rows_v.at[b], acc_sh.at[idxd_v.at[j]], ssem,
                             add=True)

        def swait(j, b):
            pltpu.make_async_copy(rows_v.at[b], acc_sh.at[idxd_v.at[j]],
                                  ssem).wait()

        nit = CHUNKS_ALL // NSLOT
        for b in range(NSLOT):
            gstart(b, b)

        def body(k, carry):
            j0 = NSLOT * k
            for b in range(NSLOT):
                gwait(j0 + b, b)
                sstart(j0 + b, b)
            for b in range(NSLOT):
                swait(j0 + b, b)

            @pl.when(k < nit - 1)
            def _():
                for b in range(NSLOT):
                    gstart(j0 + NSLOT + b, b)

            return carry

        lax.fori_loop(0, nit, body, 0)
        plsc.subcore_barrier()
        for i in range(RPT // CHUNK):
            pltpu.sync_copy(acc_sh.at[pl.ds(s * RPT + i * CHUNK, CHUNK)],
                            rows_v.at[0])

            @pl.when(c == 0)
            def _():
                pltpu.sync_copy(rows_v.at[0],
                                outa_hbm.at[pl.ds(s * RPT + i * CHUNK, CHUNK)])

            @pl.when(c == 1)
            def _():
                pltpu.sync_copy(rows_v.at[0],
                                outb_hbm.at[pl.ds(s * RPT + i * CHUNK, CHUNK)])

    return _agg


def _make_agg(D, NSLOT):
    @functools.partial(
        pl.kernel,
        out_type=jax.ShapeDtypeStruct((2 * NP, D), jnp.float32),
        mesh=_mesh,
        compiler_params=pltpu.CompilerParams(use_tc_tiling_on_sc=False),
        scratch_types=(
            [pltpu.VMEM((CHUNKS, CHUNK), jnp.int32),
             pltpu.VMEM((CHUNKS, CHUNK), jnp.int32),
             pltpu.VMEM((NSLOT, CHUNK, D), jnp.float32),
             pltpu.VMEM_SHARED((NP, D), jnp.float32),
             pltpu.VMEM_SHARED((NP, D), jnp.float32)]
            + [pltpu.SemaphoreType.DMA] * NSLOT
            + [pltpu.SemaphoreType.DMA]
        ),
    )
    def _agg(g_hbm, src_hbm, dst_hbm, zeros_hbm, out_hbm,
             idxs_v, idxd_v, rows_v, acc_sh, g_sh, *sems):
        gsems = sems[:NSLOT]
        ssem = sems[NSLOT]
        c = lax.axis_index("c")
        s = lax.axis_index("s")
        wid = s * 2 + c
        pltpu.sync_copy(g_hbm.at[pl.ds(s * RPT, RPT)],
                        g_sh.at[pl.ds(s * RPT, RPT)])
        for i in range(RPT // CHUNK):
            pltpu.sync_copy(zeros_hbm.at[pl.ds(s * RPT + i * CHUNK, CHUNK)],
                            rows_v.at[0])
            pltpu.sync_copy(rows_v.at[0],
                            acc_sh.at[pl.ds(s * RPT + i * CHUNK, CHUNK)])
        pltpu.sync_copy(src_hbm.at[pl.ds(wid * CHUNKS, CHUNKS)], idxs_v)
        pltpu.sync_copy(dst_hbm.at[pl.ds(wid * CHUNKS, CHUNKS)], idxd_v)
        plsc.subcore_barrier()

        def gstart(j, b):
            pltpu.async_copy(g_sh.at[idxs_v.at[j]], rows_v.at[b], gsems[b])

        def gwait(j, b):
            pltpu.make_async_copy(g_sh.at[idxs_v.at[j]], rows_v.at[b],
                                  gsems[b]).wait()

        def sstart(j, b):
            pltpu.async_copy(rows_v.at[b], acc_sh.at[idxd_v.at[j]], ssem,
                             add=True)

        def swait(j, b):
            pltpu.make_async_copy(rows_v.at[b], acc_sh.at[idxd_v.at[j]],
                                  ssem).wait()

        nit = CHUNKS // NSLOT
        for b in range(NSLOT):
            gstart(b, b)

        def body(k, carry):
            j0 = NSLOT * k
            for b in range(NSLOT):
                gwait(j0 + b, b)
                sstart(j0 + b, b)
            for b in range(NSLOT):
                swait(j0 + b, b)

            @pl.when(k < nit - 1)
            def _():
                for b in range(NSLOT):
                    gstart(j0 + NSLOT + b, b)

            return carry

        lax.fori_loop(0, nit, body, 0)
        plsc.subcore_barrier()
        for i in range(RPT // CHUNK):
            pltpu.sync_copy(acc_sh.at[pl.ds(s * RPT + i * CHUNK, CHUNK)],
                            rows_v.at[0])
            pltpu.sync_copy(rows_v.at[0],
                            out_hbm.at[pl.ds(c * NP + s * RPT + i * CHUNK,
                                             CHUNK)])

    return _agg


_agg_halves32 = _make_agg_halves(32, 8)
_agg8 = _make_agg(8, 8)


_B = 2048
_BR = _B // CHUNK


def _tc1_body(x_ref, w1_ref, degp_ref, g1a_ref, g1b_ref, dinv_ref):
    deg = degp_ref[0] + degp_ref[1] + 1.0
    dinv = lax.rsqrt(deg)
    dinv_ref[...] = dinv
    h = jnp.dot(x_ref[...], w1_ref[...], preferred_element_type=jnp.float32)
    g1 = (h * dinv).astype(jnp.bfloat16)
    g1a_ref[...] = g1[:, :32]
    g1b_ref[...] = g1[:, 32:]


def _tc1(x_p, W1, degp3):
    return pl.pallas_call(
        _tc1_body,
        grid=(NP // _B,),
        in_specs=[
            pl.BlockSpec((_B, 128), lambda i: (i, 0)),
            pl.BlockSpec((128, 64), lambda i: (0, 0)),
            pl.BlockSpec((2, _B, 1), lambda i: (0, i, 0)),
        ],
        out_specs=[
            pl.BlockSpec((_B, 32), lambda i: (i, 0)),
            pl.BlockSpec((_B, 32), lambda i: (i, 0)),
            pl.BlockSpec((_B, 1), lambda i: (i, 0)),
        ],
        out_shape=[
            jax.ShapeDtypeStruct((NP, 32), jnp.bfloat16),
            jax.ShapeDtypeStruct((NP, 32), jnp.bfloat16),
            jax.ShapeDtypeStruct((NP, 1), jnp.float32),
        ],
    )(x_p, W1, degp3)


def _tc2_body(p1a_ref, p1b_ref, g1a_ref, g1b_ref, dinv_ref, b1_ref, w2_ref,
              g2_ref):
    dinv = dinv_ref[...]
    sa = (p1a_ref[...] + g1a_ref[...]).astype(jnp.float32)
    sb = (p1b_ref[...] + g1b_ref[...]).astype(jnp.float32)
    ssum = jnp.concatenate([sa, sb], axis=1)
    out1 = ssum * dinv + b1_ref[...]
    r = jnp.maximum(out1, 0.0)
    h2 = jnp.dot(r, w2_ref[...], preferred_element_type=jnp.float32)
    g2_ref[...] = h2 * dinv


def _tc2(p1a, p1b, g1a, g1b, dinv, b1r, W2):
    return pl.pallas_call(
        _tc2_body,
        grid=(NP // _B,),
        in_specs=[
            pl.BlockSpec((_B, 32), lambda i: (i, 0)),
            pl.BlockSpec((_B, 32), lambda i: (i, 0)),
            pl.BlockSpec((_B, 32), lambda i: (i, 0)),
            pl.BlockSpec((_B, 32), lambda i: (i, 0)),
            pl.BlockSpec((_B, 1), lambda i: (i, 0)),
            pl.BlockSpec((1, 64), lambda i: (0, 0)),
            pl.BlockSpec((64, 8), lambda i: (0, 0)),
        ],
        out_specs=pl.BlockSpec((_B, 8), lambda i: (i, 0)),
        out_shape=jax.ShapeDtypeStruct((NP, 8), jnp.float32),
    )(p1a, p1b, g1a, g1b, dinv, b1r, W2)


def _tc3_body(p2_ref, g2_ref, dinv_ref, b2_ref, out_ref):
    dinv = dinv_ref[...]
    ssum = p2_ref[0] + p2_ref[1] + g2_ref[...]
    out_ref[...] = ssum[:, :2] * dinv + b2_ref[...]


_B3 = 2000


def _tc3(p2, g2, dinv, b2r):
    return pl.pallas_call(
        _tc3_body,
        grid=(N // _B3,),
        in_specs=[
            pl.BlockSpec((2, _B3, 8), lambda i: (0, i, 0)),
            pl.BlockSpec((_B3, 8), lambda i: (i, 0)),
            pl.BlockSpec((_B3, 1), lambda i: (i, 0)),
            pl.BlockSpec((1, 2), lambda i: (0, 0)),
        ],
        out_specs=pl.BlockSpec((_B3, 2), lambda i: (i, 0)),
        out_shape=jax.ShapeDtypeStruct((N, 2), jnp.float32),
    )(p2, g2, dinv, b2r)


def kernel(x, edge_index, W1, b1, W2, b2):
    ei = edge_index.astype(jnp.int32)
    em = jnp.pad(ei, ((0, 0), (0, EP - E)),
                 constant_values=N).reshape(2, ROWS, CHUNK)
    src_m = em[0]
    dst_m = em[1]
    x_p = jnp.pad(x, ((0, NP - N), (0, 0)))

    zeros1 = jnp.zeros((NP,), jnp.float32)
    zeros32 = jnp.zeros((NP, 32), jnp.bfloat16)
    zeros8 = jnp.zeros((NP, 8), jnp.float32)
    W2p = jnp.pad(W2, ((0, 0), (0, 8 - 2)))

    degp3 = _deg_kernel(dst_m, zeros1).reshape(2, NP, 1)
    g1a, g1b, dinv = _tc1(x_p, W1, degp3)
    p1a, p1b = _agg_halves32(g1a, g1b, src_m, dst_m, zeros32)
    g2 = _tc2(p1a, p1b, g1a, g1b, dinv, b1.reshape(1, 64), W2p)
    p2 = _agg8(g2, src_m, dst_m, zeros8).reshape(2, NP, 8)
    return _tc3(p2, g2, dinv, b2.reshape(1, 2))
